# baseline (speedup 1.0000x reference)
import jax
import jax.numpy as jnp
from jax import lax
from jax.experimental import pallas as pl
from jax.experimental.pallas import tpu as pltpu
from jax.experimental.pallas import tpu_sc as plsc

EMB = 16
MARGIN = 0.1
NC = 2
NS = 16
NW = NC * NS
TERMS = 32768
TPW = TERMS // NW
CHUNK = 128
NCH = TPW // CHUNK
BLOCKS = TPW // 16


def _sc_body(idx_hbm, ent_hbm, rel_hbm, out_hbm,
             idx_v, hp, rp, tp, hn, rn, tn, sqp_v, sqn_v, sem):
    wid = lax.axis_index("s") * NC + lax.axis_index("c")
    pltpu.sync_copy(idx_hbm.at[wid], idx_v)

    bufs = (hp, rp, tp, hn, rn, tn)
    tables = (ent_hbm, rel_hbm, ent_hbm, ent_hbm, rel_hbm, ent_hbm)
    copies = []
    for j in range(6):
        for c in range(NCH):
            copies.append(pltpu.async_copy(
                tables[j].at[idx_v.at[j, c]],
                bufs[j].at[pl.ds(c * CHUNK, CHUNK)], sem))
    for cp in copies:
        cp.wait()

    iot = lax.iota(jnp.int32, 16)
    perms = [iot ^ s for s in (8, 4, 2, 1)]
    masks = [(iot & s) == 0 for s in (8, 4, 2, 1)]

    dnums = lax.GatherDimensionNumbers(
        offset_dims=(), collapsed_slice_dims=(0,), start_index_map=(0,))

    def _perm(v, pidx):
        return lax.gather(v, pidx[:, None], dnums, (1,),
                          mode=lax.GatherScatterMode.PROMISE_IN_BOUNDS)

    def _rowsums(h, r, t, base):
        es = []
        for j in range(16):
            d = h[base + j, :] + r[base + j, :] - t[base + j, :]
            es.append(d * d)
        for pidx, msk in zip(perms, masks):
            half = len(es) // 2
            es = [jnp.where(msk,
                            es[i] + _perm(es[i], pidx),
                            es[i + half] + _perm(es[i + half], pidx))
                  for i in range(half)]
        return es[0]

    def block(b, carry):
        base = b * 16
        sqp_v[pl.ds(base, 16)] = _rowsums(hp, rp, tp, base)
        sqn_v[pl.ds(base, 16)] = _rowsums(hn, rn, tn, base)
        return carry

    lax.fori_loop(0, BLOCKS, block, jnp.int32(0))
    pltpu.sync_copy(sqp_v, out_hbm.at[0, pl.ds(wid * TPW, TPW)])
    pltpu.sync_copy(sqn_v, out_hbm.at[1, pl.ds(wid * TPW, TPW)])


TILE_COLS = 7813
PAD_ENT = TILE_COLS * 128
GT = 4
GC = GT * 128
FULLG = 1952
TAILC = 576
GPAIRS = 31


def _relayout_body(te_hbm, tr_hbm, oe_hbm, or_hbm,
                   ia0, ia1, oba, obb, tin0, sem, osem):
    wid = lax.axis_index("s") * NC + lax.axis_index("c")
    i16 = lax.iota(jnp.int32, 16) * 16
    ins = (ia0, ia1)
    obs = (oba, obb)

    def issue_in(tab, g, b):
        c0 = g * GC
        pltpu.async_copy(tab.at[pl.ds(0, 16), pl.ds(c0, GC)], ins[b], sem)

    def wait_in(tab, b):
        pltpu.make_async_copy(tab.at[pl.ds(0, 16), pl.ds(0, GC)],
                              ins[b], sem).wait()

    def transpose_tile(src, dst, tc, jmax):
        for j in range(jmax):
            for k in range(16):
                plsc.store_scatter(dst, [i16 + (tc * 2048 + j * 256 + k)],
                                   src[k, pl.ds(tc * 128 + j * 16, 16)])

    def do_table(tab, out):
        issue_in(tab, wid, 0)
        issue_in(tab, wid + NW, 1)

        def pair(s2, carry):
            for b in (0, 1):
                g = wid + (2 * s2 + b) * NW

                @pl.when(g < FULLG)
                def _():
                    wait_in(tab, b)

                    @pl.when(2 * s2 + b >= 2)
                    def _():
                        pltpu.make_async_copy(
                            obs[b].at[pl.ds(0, GC * EMB)],
                            out.at[pl.ds(0, GC * EMB)], osem).wait()
                    for tc in range(GT):
                        transpose_tile(ins[b], obs[b], tc, 8)
                    gp = g + 2 * NW

                    @pl.when(gp < FULLG)
                    def _():
                        issue_in(tab, gp, b)
                    pltpu.async_copy(obs[b].at[pl.ds(0, GC * EMB)],
                                     out.at[pl.ds(g * GC * EMB, GC * EMB)],
                                     osem)
            return carry

        lax.fori_loop(0, GPAIRS, pair, jnp.int32(0))
        for b in (0, 1):
            pltpu.make_async_copy(obs[b].at[pl.ds(0, GC * EMB)],
                                  out.at[pl.ds(0, GC * EMB)], osem).wait()

    def do_tail(tab, out):
        c0 = FULLG * GC
        pltpu.sync_copy(tab.at[pl.ds(0, 16), pl.ds(c0, TAILC)], tin0)
        for tc in range(4):
            transpose_tile(tin0, oba, tc, 8)
        transpose_tile(tin0, oba, 4, 4)
        pltpu.sync_copy(oba.at[pl.ds(0, TAILC * EMB)],
                        out.at[pl.ds(c0 * EMB, TAILC * EMB)])

    do_table(te_hbm, oe_hbm)
    do_table(tr_hbm, or_hbm)

    @pl.when(wid == 31)
    def _():
        do_tail(te_hbm, oe_hbm)
        do_tail(tr_hbm, or_hbm)


def _loss_body(x_ref, o_ref):
    sp = x_ref[0, :]
    sn = x_ref[1, :]
    loss = jnp.maximum(MARGIN + jnp.sqrt(sp) - jnp.sqrt(sn), 0.0)
    o_ref[...] = jnp.sum(loss).reshape(1, 1)


def kernel(lhs_pos, rhs_pos, lhs_neg, rhs_neg, ent_emb, rel_emb):
    pos = jnp.concatenate([lhs_pos, rhs_pos], axis=0).astype(jnp.int32)
    neg = jnp.concatenate([lhs_neg, rhs_neg], axis=0).astype(jnp.int32)
    allidx = jnp.stack([pos[:, 0], pos[:, 1], pos[:, 2],
                        neg[:, 0], neg[:, 1], neg[:, 2]])
    idx = (allidx.reshape(6, NW, TPW).transpose(1, 0, 2)
           .reshape(NW, 6, NCH, CHUNK))

    mesh = plsc.VectorSubcoreMesh(core_axis_name="c", subcore_axis_name="s")
    relay = pl.kernel(
        _relayout_body,
        out_type=[jax.ShapeDtypeStruct((PAD_ENT * EMB,), jnp.float32),
                  jax.ShapeDtypeStruct((PAD_ENT * EMB,), jnp.float32)],
        mesh=mesh,
        scratch_types=[
            pltpu.VMEM((16, GC), jnp.float32),
            pltpu.VMEM((16, GC), jnp.float32),
            pltpu.VMEM((TAILC * EMB,), jnp.float32),
            pltpu.VMEM((GC * EMB,), jnp.float32),
            pltpu.VMEM((16, TAILC), jnp.float32),
            pltpu.SemaphoreType.DMA,
            pltpu.SemaphoreType.DMA,
        ],
        compiler_params=pltpu.CompilerParams(use_tc_tiling_on_sc=True,
                                             needs_layout_passes=False),
    )
    flat_e, flat_r = relay(ent_emb.T, rel_emb.T)
    lin_e = flat_e.reshape(PAD_ENT, EMB)
    lin_r = flat_r.reshape(PAD_ENT, EMB)
    sc = pl.kernel(
        _sc_body,
        out_type=jax.ShapeDtypeStruct((2, TERMS), jnp.float32),
        mesh=mesh,
        scratch_types=[
            pltpu.VMEM((6, NCH, CHUNK), jnp.int32),
            pltpu.VMEM((TPW, EMB), jnp.float32),
            pltpu.VMEM((TPW, EMB), jnp.float32),
            pltpu.VMEM((TPW, EMB), jnp.float32),
            pltpu.VMEM((TPW, EMB), jnp.float32),
            pltpu.VMEM((TPW, EMB), jnp.float32),
            pltpu.VMEM((TPW, EMB), jnp.float32),
            pltpu.VMEM((TPW,), jnp.float32),
            pltpu.VMEM((TPW,), jnp.float32),
            pltpu.SemaphoreType.DMA,
        ],
        compiler_params=pltpu.CompilerParams(use_tc_tiling_on_sc=False,
                                             needs_layout_passes=False),
    )
    sq = sc(idx, lin_e, lin_r)

    loss = pl.pallas_call(
        _loss_body,
        out_shape=jax.ShapeDtypeStruct((1, 1), jnp.float32),
    )(sq)
    return loss[0, 0]


# speedup vs baseline: 1.8511x; 1.8511x over previous
import jax
import jax.numpy as jnp
from jax import lax
from jax.experimental import pallas as pl
from jax.experimental.pallas import tpu as pltpu
from jax.experimental.pallas import tpu_sc as plsc

EMB = 16
MARGIN = 0.1
NC = 2
NS = 16
NW = NC * NS
TERMS = 32768
TPW = TERMS // NW
CHUNK = 128
NCH = TPW // CHUNK
BLOCKS = TPW // 16


_DNUMS = lax.GatherDimensionNumbers(
    offset_dims=(), collapsed_slice_dims=(0,), start_index_map=(0,))


def _lane_perm(v, pidx):
    return lax.gather(v, pidx[:, None], _DNUMS, (1,),
                      mode=lax.GatherScatterMode.PROMISE_IN_BOUNDS)


def _sc_body(idx_hbm, ent_hbm, rel_hbm, out_hbm,
             idx_v, hp, rp, tp, hn, rn, tn, sqp_v, sqn_v, sem):
    wid = lax.axis_index("s") * NC + lax.axis_index("c")
    pltpu.sync_copy(idx_hbm.at[wid], idx_v)

    bufs = (hp, rp, tp, hn, rn, tn)
    tables = (ent_hbm, rel_hbm, ent_hbm, ent_hbm, rel_hbm, ent_hbm)
    copies = []
    for j in range(6):
        for c in range(NCH):
            copies.append(pltpu.async_copy(
                tables[j].at[idx_v.at[j, c]],
                bufs[j].at[pl.ds(c * CHUNK, CHUNK)], sem))
    for cp in copies:
        cp.wait()

    iot = lax.iota(jnp.int32, 16)
    perms = [iot ^ s for s in (8, 4, 2, 1)]
    masks = [(iot & s) == 0 for s in (8, 4, 2, 1)]

    _perm = _lane_perm

    def _rowsums(h, r, t, base):
        es = []
        for j in range(16):
            d = h[base + j, :] + r[base + j, :] - t[base + j, :]
            es.append(d * d)
        for pidx, msk in zip(perms, masks):
            half = len(es) // 2
            es = [jnp.where(msk,
                            es[i] + _perm(es[i], pidx),
                            es[i + half] + _perm(es[i + half], pidx))
                  for i in range(half)]
        return es[0]

    def block(b, carry):
        base = b * 16
        sqp_v[pl.ds(base, 16)] = _rowsums(hp, rp, tp, base)
        sqn_v[pl.ds(base, 16)] = _rowsums(hn, rn, tn, base)
        return carry

    lax.fori_loop(0, BLOCKS, block, jnp.int32(0))
    pltpu.sync_copy(sqp_v, out_hbm.at[0, pl.ds(wid * TPW, TPW)])
    pltpu.sync_copy(sqn_v, out_hbm.at[1, pl.ds(wid * TPW, TPW)])


TILE_COLS = 7813
PAD_ENT = TILE_COLS * 128
GT = 4
GC = GT * 128
FULLG = 1952
TAILC = 576
GPAIRS = 31


def _relayout_body(te_hbm, tr_hbm, oe_hbm, or_hbm,
                   ia0, ia1, oba, obb, tin0, sem, osem):
    wid = lax.axis_index("s") * NC + lax.axis_index("c")
    iot = lax.iota(jnp.int32, 16)
    tperms = {s_: iot ^ s_ for s_ in (8, 4, 2, 1)}
    tmasks = {s_: (iot & s_) == 0 for s_ in (8, 4, 2, 1)}
    ins = (ia0, ia1)
    obs = (oba, obb)

    def issue_in(tab, g, b):
        c0 = g * GC
        pltpu.async_copy(tab.at[pl.ds(0, 16), pl.ds(c0, GC)], ins[b], sem)

    def wait_in(tab, b):
        pltpu.make_async_copy(tab.at[pl.ds(0, 16), pl.ds(0, GC)],
                              ins[b], sem).wait()

    def transpose_block(src, dst, blk):
        col0 = blk * 16
        vs = [src[k, pl.ds(col0, 16)] for k in range(16)]
        for s_ in (8, 4, 2, 1):
            nv = list(vs)
            for i in range(16):
                if i & s_ == 0:
                    a, b = vs[i], vs[i + s_]
                    nv[i] = jnp.where(tmasks[s_], a, _lane_perm(b, tperms[s_]))
                    nv[i + s_] = jnp.where(tmasks[s_], _lane_perm(a, tperms[s_]), b)
            vs = nv
        for e in range(16):
            dst[pl.ds(blk * 256 + e * 16, 16)] = vs[e]

    def transpose_range(src, dst, nblk):
        def blkfn(blk, carry):
            transpose_block(src, dst, blk)
            return carry
        lax.fori_loop(0, nblk, blkfn, jnp.int32(0))

    def do_table(tab, out):
        issue_in(tab, wid, 0)
        issue_in(tab, wid + NW, 1)

        def pair(s2, carry):
            for b in (0, 1):
                g = wid + (2 * s2 + b) * NW

                @pl.when(g < FULLG)
                def _():
                    wait_in(tab, b)

                    @pl.when(2 * s2 + b >= 2)
                    def _():
                        pltpu.make_async_copy(
                            obs[b].at[pl.ds(0, GC * EMB)],
                            out.at[pl.ds(0, GC * EMB)], osem).wait()
                    transpose_range(ins[b], obs[b], GC // 16)
                    gp = g + 2 * NW

                    @pl.when(gp < FULLG)
                    def _():
                        issue_in(tab, gp, b)
                    pltpu.async_copy(obs[b].at[pl.ds(0, GC * EMB)],
                                     out.at[pl.ds(g * GC * EMB, GC * EMB)],
                                     osem)
            return carry

        lax.fori_loop(0, GPAIRS, pair, jnp.int32(0))
        for b in (0, 1):
            pltpu.make_async_copy(obs[b].at[pl.ds(0, GC * EMB)],
                                  out.at[pl.ds(0, GC * EMB)], osem).wait()

    def do_tail(tab, out):
        c0 = FULLG * GC
        pltpu.sync_copy(tab.at[pl.ds(0, 16), pl.ds(c0, TAILC)], tin0)
        transpose_range(tin0, oba, TAILC // 16)
        pltpu.sync_copy(oba.at[pl.ds(0, TAILC * EMB)],
                        out.at[pl.ds(c0 * EMB, TAILC * EMB)])

    do_table(te_hbm, oe_hbm)
    do_table(tr_hbm, or_hbm)

    @pl.when(wid == 31)
    def _():
        do_tail(te_hbm, oe_hbm)
        do_tail(tr_hbm, or_hbm)


def _loss_body(x_ref, o_ref):
    sp = x_ref[0, :]
    sn = x_ref[1, :]
    loss = jnp.maximum(MARGIN + jnp.sqrt(sp) - jnp.sqrt(sn), 0.0)
    o_ref[...] = jnp.sum(loss).reshape(1, 1)


def kernel(lhs_pos, rhs_pos, lhs_neg, rhs_neg, ent_emb, rel_emb):
    pos = jnp.concatenate([lhs_pos, rhs_pos], axis=0).astype(jnp.int32)
    neg = jnp.concatenate([lhs_neg, rhs_neg], axis=0).astype(jnp.int32)
    allidx = jnp.stack([pos[:, 0], pos[:, 1], pos[:, 2],
                        neg[:, 0], neg[:, 1], neg[:, 2]])
    idx = (allidx.reshape(6, NW, TPW).transpose(1, 0, 2)
           .reshape(NW, 6, NCH, CHUNK))

    mesh = plsc.VectorSubcoreMesh(core_axis_name="c", subcore_axis_name="s")
    relay = pl.kernel(
        _relayout_body,
        out_type=[jax.ShapeDtypeStruct((PAD_ENT * EMB,), jnp.float32),
                  jax.ShapeDtypeStruct((PAD_ENT * EMB,), jnp.float32)],
        mesh=mesh,
        scratch_types=[
            pltpu.VMEM((16, GC), jnp.float32),
            pltpu.VMEM((16, GC), jnp.float32),
            pltpu.VMEM((TAILC * EMB,), jnp.float32),
            pltpu.VMEM((GC * EMB,), jnp.float32),
            pltpu.VMEM((16, TAILC), jnp.float32),
            pltpu.SemaphoreType.DMA,
            pltpu.SemaphoreType.DMA,
        ],
        compiler_params=pltpu.CompilerParams(use_tc_tiling_on_sc=True,
                                             needs_layout_passes=False),
    )
    flat_e, flat_r = relay(ent_emb.T, rel_emb.T)
    lin_e = flat_e.reshape(PAD_ENT, EMB)
    lin_r = flat_r.reshape(PAD_ENT, EMB)
    sc = pl.kernel(
        _sc_body,
        out_type=jax.ShapeDtypeStruct((2, TERMS), jnp.float32),
        mesh=mesh,
        scratch_types=[
            pltpu.VMEM((6, NCH, CHUNK), jnp.int32),
            pltpu.VMEM((TPW, EMB), jnp.float32),
            pltpu.VMEM((TPW, EMB), jnp.float32),
            pltpu.VMEM((TPW, EMB), jnp.float32),
            pltpu.VMEM((TPW, EMB), jnp.float32),
            pltpu.VMEM((TPW, EMB), jnp.float32),
            pltpu.VMEM((TPW, EMB), jnp.float32),
            pltpu.VMEM((TPW,), jnp.float32),
            pltpu.VMEM((TPW,), jnp.float32),
            pltpu.SemaphoreType.DMA,
        ],
        compiler_params=pltpu.CompilerParams(use_tc_tiling_on_sc=False,
                                             needs_layout_passes=False),
    )
    sq = sc(idx, lin_e, lin_r)

    loss = pl.pallas_call(
        _loss_body,
        out_shape=jax.ShapeDtypeStruct((1, 1), jnp.float32),
    )(sq)
    return loss[0, 0]


# speedup vs baseline: 1.9462x; 1.0514x over previous
import jax
import jax.numpy as jnp
from jax import lax
from jax.experimental import pallas as pl
from jax.experimental.pallas import tpu as pltpu
from jax.experimental.pallas import tpu_sc as plsc

EMB = 16
MARGIN = 0.1
NC = 2
NS = 16
NW = NC * NS
TERMS = 32768
TPW = TERMS // NW
CHUNK = 128
NCH = TPW // CHUNK
BLOCKS = TPW // 16


_DNUMS = lax.GatherDimensionNumbers(
    offset_dims=(), collapsed_slice_dims=(0,), start_index_map=(0,))


def _lane_perm(v, pidx):
    return lax.gather(v, pidx[:, None], _DNUMS, (1,),
                      mode=lax.GatherScatterMode.PROMISE_IN_BOUNDS)


def _sc_body(idx_hbm, ent_hbm, rel_hbm, out_hbm,
             idx_v, hp, rp, tp, hn, rn, tn, sqp_v, sqn_v, sem):
    wid = lax.axis_index("s") * NC + lax.axis_index("c")
    pltpu.sync_copy(idx_hbm.at[wid], idx_v)

    bufs = (hp, rp, tp, hn, rn, tn)
    tables = (ent_hbm, rel_hbm, ent_hbm, ent_hbm, rel_hbm, ent_hbm)
    copies = []
    for j in range(6):
        for c in range(NCH):
            copies.append(pltpu.async_copy(
                tables[j].at[idx_v.at[j, c]],
                bufs[j].at[pl.ds(c * CHUNK, CHUNK)], sem))
    for cp in copies:
        cp.wait()

    iot = lax.iota(jnp.int32, 16)
    perms = [iot ^ s for s in (8, 4, 2, 1)]
    masks = [(iot & s) == 0 for s in (8, 4, 2, 1)]

    _perm = _lane_perm

    def _rowsums(h, r, t, base):
        es = []
        for j in range(16):
            d = h[base + j, :] + r[base + j, :] - t[base + j, :]
            es.append(d * d)
        for pidx, msk in zip(perms, masks):
            half = len(es) // 2
            es = [jnp.where(msk,
                            es[i] + _perm(es[i], pidx),
                            es[i + half] + _perm(es[i + half], pidx))
                  for i in range(half)]
        return es[0]

    def block(b, carry):
        base = b * 16
        sqp_v[pl.ds(base, 16)] = _rowsums(hp, rp, tp, base)
        sqn_v[pl.ds(base, 16)] = _rowsums(hn, rn, tn, base)
        return carry

    lax.fori_loop(0, BLOCKS, block, jnp.int32(0))
    pltpu.sync_copy(sqp_v, out_hbm.at[0, pl.ds(wid * TPW, TPW)])
    pltpu.sync_copy(sqn_v, out_hbm.at[1, pl.ds(wid * TPW, TPW)])


TILE_COLS = 7813
PAD_ENT = TILE_COLS * 128
GT = 8
GC = GT * 128
FULLG = 976
TAILC = 576
GPAIRS = 16


def _relayout_body(te_hbm, tr_hbm, oe_hbm, or_hbm,
                   ia0, ia1, oba, obb, tin0, sem, osem):
    wid = lax.axis_index("s") * NC + lax.axis_index("c")
    iot = lax.iota(jnp.int32, 16)
    tperms = {s_: iot ^ s_ for s_ in (8, 4, 2, 1)}
    tmasks = {s_: (iot & s_) == 0 for s_ in (8, 4, 2, 1)}
    ins = (ia0, ia1)
    obs = (oba, obb)

    def issue_in(tab, g, b):
        c0 = g * GC
        pltpu.async_copy(tab.at[pl.ds(0, 16), pl.ds(c0, GC)], ins[b], sem)

    def wait_in(tab, b):
        pltpu.make_async_copy(tab.at[pl.ds(0, 16), pl.ds(0, GC)],
                              ins[b], sem).wait()

    def transpose_block(src, dst, blk):
        col0 = blk * 16
        vs = [src[k, pl.ds(col0, 16)] for k in range(16)]
        for s_ in (8, 4, 2, 1):
            nv = list(vs)
            for i in range(16):
                if i & s_ == 0:
                    a, b = vs[i], vs[i + s_]
                    nv[i] = jnp.where(tmasks[s_], a, _lane_perm(b, tperms[s_]))
                    nv[i + s_] = jnp.where(tmasks[s_], _lane_perm(a, tperms[s_]), b)
            vs = nv
        for e in range(16):
            dst[pl.ds(blk * 256 + e * 16, 16)] = vs[e]

    def transpose_range(src, dst, nblk):
        def blkfn(blk, carry):
            transpose_block(src, dst, blk)
            return carry
        lax.fori_loop(0, nblk, blkfn, jnp.int32(0))

    def do_table(tab, out):
        issue_in(tab, wid, 0)
        issue_in(tab, wid + NW, 1)

        def pair(s2, carry):
            for b in (0, 1):
                g = wid + (2 * s2 + b) * NW

                @pl.when(g < FULLG)
                def _():
                    wait_in(tab, b)

                    @pl.when(2 * s2 + b >= 2)
                    def _():
                        pltpu.make_async_copy(
                            obs[b].at[pl.ds(0, GC * EMB)],
                            out.at[pl.ds(0, GC * EMB)], osem).wait()
                    transpose_range(ins[b], obs[b], GC // 16)
                    gp = g + 2 * NW

                    @pl.when(gp < FULLG)
                    def _():
                        issue_in(tab, gp, b)
                    pltpu.async_copy(obs[b].at[pl.ds(0, GC * EMB)],
                                     out.at[pl.ds(g * GC * EMB, GC * EMB)],
                                     osem)
            return carry

        lax.fori_loop(0, GPAIRS, pair, jnp.int32(0))
        for b in (0, 1):
            pltpu.make_async_copy(obs[b].at[pl.ds(0, GC * EMB)],
                                  out.at[pl.ds(0, GC * EMB)], osem).wait()

    def do_tail(tab, out):
        c0 = FULLG * GC
        pltpu.sync_copy(tab.at[pl.ds(0, 16), pl.ds(c0, TAILC)], tin0)
        transpose_range(tin0, oba, TAILC // 16)
        pltpu.sync_copy(oba.at[pl.ds(0, TAILC * EMB)],
                        out.at[pl.ds(c0 * EMB, TAILC * EMB)])

    do_table(te_hbm, oe_hbm)
    do_table(tr_hbm, or_hbm)

    @pl.when(wid == 31)
    def _():
        do_tail(te_hbm, oe_hbm)
        do_tail(tr_hbm, or_hbm)


def _loss_body(x_ref, o_ref):
    sp = x_ref[0, :]
    sn = x_ref[1, :]
    loss = jnp.maximum(MARGIN + jnp.sqrt(sp) - jnp.sqrt(sn), 0.0)
    o_ref[...] = jnp.sum(loss).reshape(1, 1)


def kernel(lhs_pos, rhs_pos, lhs_neg, rhs_neg, ent_emb, rel_emb):
    pos = jnp.concatenate([lhs_pos, rhs_pos], axis=0).astype(jnp.int32)
    neg = jnp.concatenate([lhs_neg, rhs_neg], axis=0).astype(jnp.int32)
    allidx = jnp.stack([pos[:, 0], pos[:, 1], pos[:, 2],
                        neg[:, 0], neg[:, 1], neg[:, 2]])
    idx = (allidx.reshape(6, NW, TPW).transpose(1, 0, 2)
           .reshape(NW, 6, NCH, CHUNK))

    mesh = plsc.VectorSubcoreMesh(core_axis_name="c", subcore_axis_name="s")
    relay = pl.kernel(
        _relayout_body,
        out_type=[jax.ShapeDtypeStruct((PAD_ENT * EMB,), jnp.float32),
                  jax.ShapeDtypeStruct((PAD_ENT * EMB,), jnp.float32)],
        mesh=mesh,
        scratch_types=[
            pltpu.VMEM((16, GC), jnp.float32),
            pltpu.VMEM((16, GC), jnp.float32),
            pltpu.VMEM((TAILC * EMB,), jnp.float32),
            pltpu.VMEM((GC * EMB,), jnp.float32),
            pltpu.VMEM((16, TAILC), jnp.float32),
            pltpu.SemaphoreType.DMA,
            pltpu.SemaphoreType.DMA,
        ],
        compiler_params=pltpu.CompilerParams(use_tc_tiling_on_sc=True,
                                             needs_layout_passes=False),
    )
    flat_e, flat_r = relay(ent_emb.T, rel_emb.T)
    lin_e = flat_e.reshape(PAD_ENT, EMB)
    lin_r = flat_r.reshape(PAD_ENT, EMB)
    sc = pl.kernel(
        _sc_body,
        out_type=jax.ShapeDtypeStruct((2, TERMS), jnp.float32),
        mesh=mesh,
        scratch_types=[
            pltpu.VMEM((6, NCH, CHUNK), jnp.int32),
            pltpu.VMEM((TPW, EMB), jnp.float32),
            pltpu.VMEM((TPW, EMB), jnp.float32),
            pltpu.VMEM((TPW, EMB), jnp.float32),
            pltpu.VMEM((TPW, EMB), jnp.float32),
            pltpu.VMEM((TPW, EMB), jnp.float32),
            pltpu.VMEM((TPW, EMB), jnp.float32),
            pltpu.VMEM((TPW,), jnp.float32),
            pltpu.VMEM((TPW,), jnp.float32),
            pltpu.SemaphoreType.DMA,
        ],
        compiler_params=pltpu.CompilerParams(use_tc_tiling_on_sc=False,
                                             needs_layout_passes=False),
    )
    sq = sc(idx, lin_e, lin_r)

    loss = pl.pallas_call(
        _loss_body,
        out_shape=jax.ShapeDtypeStruct((1, 1), jnp.float32),
    )(sq)
    return loss[0, 0]
